# baseline (speedup 1.0000x reference)
import functools

import jax
import jax.numpy as jnp
from jax import lax
from jax.experimental import pallas as pl
from jax.experimental.pallas import tpu as pltpu
from jax.experimental.pallas import tpu_sc as plsc

_BITS = 32


def _build_sc_kernel(B, V, D, Vpad):
    info = plsc.get_sparse_core_info()
    NC, NS, L = info.num_cores, info.num_subcores, info.num_lanes
    NW = NC * NS
    assert D * _BITS == L * NW
    n_groups = B // L
    half = _BITS // 2

    mesh = plsc.VectorSubcoreMesh(core_axis_name="c", subcore_axis_name="s")

    @functools.partial(
        pl.kernel,
        mesh=mesh,
        compiler_params=pltpu.CompilerParams(needs_layout_passes=False),
        out_type=jax.ShapeDtypeStruct((D * _BITS, B), jnp.float32),
        scratch_types=[
            pltpu.VMEM((B,), jnp.int32),
            pltpu.VMEM((Vpad,), jnp.float32),
            pltpu.VMEM((L, B), jnp.float32),
            pltpu.SemaphoreType.DMA,
            pltpu.SemaphoreType.DMA,
        ],
    )
    def sc_kernel(tok_hbm, wt_hbm, out_hbm, tok_v, trow_v, out_v, sem, in_sem):
        wid = lax.axis_index("s") * NC + lax.axis_index("c")
        d = lax.shift_right_logical(wid, 1)
        kh = wid & 1
        cp_tok = pltpu.async_copy(tok_hbm, tok_v, in_sem)
        cp_row = pltpu.async_copy(wt_hbm.at[d], trow_v, in_sem)
        cp_tok.wait()
        cp_row.wait()

        k_base = kh * half

        n_chunks = 2
        g_per_chunk = n_groups // n_chunks
        b_chunk = B // n_chunks

        copies = []
        for c in range(n_chunks):

            @plsc.parallel_loop(c * g_per_chunk, (c + 1) * g_per_chunk, unroll=8)
            def grp_body(gr):
                t = tok_v[pl.ds(gr * L, L)]
                vals = plsc.load_gather(trow_v, [t])
                bits = lax.bitcast_convert_type(vals, jnp.int32)
                one_f = jnp.int32(0x3F800000)
                for j in range(half):
                    sign = lax.shift_right_arithmetic(
                        lax.shift_left(bits, k_base + j), 31
                    )
                    out_v[j, pl.ds(gr * L, L)] = lax.bitcast_convert_type(
                        sign & one_f, jnp.float32
                    )

            copies.append(
                pltpu.async_copy(
                    out_v.at[:, pl.ds(c * b_chunk, b_chunk)],
                    out_hbm.at[pl.ds(wid * L, L), pl.ds(c * b_chunk, b_chunk)],
                    sem,
                )
            )
        for cp in copies:
            cp.wait()

    return sc_kernel


def kernel(token_ids, weight_float):
    B = token_ids.shape[0]
    V, D = weight_float.shape
    Vpad = 1024 if V <= 1024 else -(-V // 128) * 128
    tok = token_ids.astype(jnp.int32)
    w_t = jnp.pad(weight_float.T, ((0, 0), (0, Vpad - V)))
    sc = _build_sc_kernel(B, V, D, Vpad)
    out2 = sc(tok, w_t)
    return jnp.transpose(out2.reshape(D, _BITS, B), (2, 0, 1))


# speedup vs baseline: 1.3801x; 1.3801x over previous
import functools

import jax
import jax.numpy as jnp
from jax import lax
from jax.experimental import pallas as pl
from jax.experimental.pallas import tpu as pltpu
from jax.experimental.pallas import tpu_sc as plsc

_BITS = 32


def _build_sc_kernel(B, V, D, Vpad):
    info = plsc.get_sparse_core_info()
    NC, NS, L = info.num_cores, info.num_subcores, info.num_lanes
    NW = NC * NS
    assert D * _BITS == L * NW
    n_groups = B // L
    half = _BITS // 2

    mesh = plsc.VectorSubcoreMesh(core_axis_name="c", subcore_axis_name="s")

    @functools.partial(
        pl.kernel,
        mesh=mesh,
        compiler_params=pltpu.CompilerParams(needs_layout_passes=False),
        out_type=jax.ShapeDtypeStruct((D * _BITS, B), jnp.float32),
        scratch_types=[
            pltpu.VMEM((B,), jnp.int32),
            pltpu.VMEM((Vpad,), jnp.float32),
            pltpu.VMEM((L, B), jnp.float32),
            pltpu.SemaphoreType.DMA,
            pltpu.SemaphoreType.DMA,
        ],
    )
    def sc_kernel(tok_hbm, wt_hbm, out_hbm, tok_v, trow_v, out_v, sem, in_sem):
        wid = lax.axis_index("s") * NC + lax.axis_index("c")
        d = lax.shift_right_logical(wid, 1)
        kh = wid & 1
        cp_tok = pltpu.async_copy(tok_hbm, tok_v, in_sem)
        cp_row = pltpu.async_copy(wt_hbm.at[d], trow_v, in_sem)
        cp_tok.wait()
        cp_row.wait()

        sh_base = 31 - kh * half

        n_chunks = 2
        g_per_chunk = n_groups // n_chunks
        b_chunk = B // n_chunks

        copies = []
        for c in range(n_chunks):

            @plsc.parallel_loop(c * g_per_chunk, (c + 1) * g_per_chunk, unroll=8)
            def grp_body(gr):
                t = tok_v[pl.ds(gr * L, L)]
                vals = plsc.load_gather(trow_v, [t])
                bits = lax.bitcast_convert_type(vals, jnp.int32)
                for j in range(half):
                    bit = lax.shift_right_logical(bits, sh_base - j) & 1
                    out_v[j, pl.ds(gr * L, L)] = bit.astype(jnp.float32)

            copies.append(
                pltpu.async_copy(
                    out_v.at[:, pl.ds(c * b_chunk, b_chunk)],
                    out_hbm.at[pl.ds(wid * L, L), pl.ds(c * b_chunk, b_chunk)],
                    sem,
                )
            )
        for cp in copies:
            cp.wait()

    return sc_kernel


def kernel(token_ids, weight_float):
    B = token_ids.shape[0]
    V, D = weight_float.shape
    Vpad = 1024 if V <= 1024 else -(-V // 128) * 128
    tok = token_ids.astype(jnp.int32)
    w_t = jnp.pad(weight_float.T, ((0, 0), (0, Vpad - V)))
    sc = _build_sc_kernel(B, V, D, Vpad)
    out2 = sc(tok, w_t)
    return jnp.transpose(out2.reshape(D, _BITS, B), (2, 0, 1))
